# baseline (speedup 1.0000x reference)
import math

import jax
import jax.numpy as jnp
from jax import lax
from jax.experimental import pallas as pl
from jax.experimental.pallas import tpu as pltpu
from jax.experimental.pallas import tpu_sc as plsc

D_MODEL = 64
VOCAB = 1000000
BATCH = 4096
SEQ = 200
SCALE = math.sqrt(D_MODEL)

NC, NS, L = 2, 16, 16
NW = NC * NS

PACK_ROWS = VOCAB // 2
NBLK_FULL = VOCAB // 128
TAIL_V0 = NBLK_FULL * 128
A_NBUF = 4
B_NBUF = 4


def _repack_body(tt_hbm, packed_hbm, *scratch):
    bufs = scratch[:A_NBUF]
    outs = scratch[A_NBUF:2 * A_NBUF]
    isem = scratch[2 * A_NBUF:3 * A_NBUF]
    osem = scratch[3 * A_NBUF:4 * A_NBUF]

    wid = lax.axis_index("s") * NC + lax.axis_index("c")
    nblk_w = (NBLK_FULL - wid + NW - 1) // NW
    lane = lax.iota(jnp.int32, 16)

    def outer(o, carry):
        for b in range(A_NBUF):
            n = o * A_NBUF + b

            @pl.when(n < nblk_w)
            def _start(b=b, n=n):
                @pl.when(o > 0)
                def _drain(b=b):
                    pltpu.make_async_copy(
                        outs[b], packed_hbm.at[pl.ds(0, 64)], osem[b]).wait()

                v0 = pl.multiple_of((wid + n * NW) * 128, 128)
                pltpu.async_copy(
                    tt_hbm.at[pl.ds(0, D_MODEL), pl.ds(v0, 128)],
                    bufs[b], isem[b])

        for b in range(A_NBUF):
            n = o * A_NBUF + b

            @pl.when(n < nblk_w)
            def _work(b=b, n=n):
                v0 = pl.multiple_of((wid + n * NW) * 128, 128)
                pltpu.make_async_copy(
                    tt_hbm.at[pl.ds(0, D_MODEL), pl.ds(v0, 128)],
                    bufs[b], isem[b]).wait()

                def pack_row(m, c2, b=b):
                    for s in range(8):
                        row_ids = lane + 16 * (s % 4)
                        col_ids = (lane * 0) + (2 * m + (s // 4))
                        vals = plsc.load_gather(bufs[b], [row_ids, col_ids])
                        outs[b][m, pl.ds(16 * s, 16)] = vals * SCALE
                    return c2

                lax.fori_loop(0, 64, pack_row, 0)
                j0 = pl.multiple_of((wid + n * NW) * 64, 8)
                pltpu.async_copy(outs[b], packed_hbm.at[pl.ds(j0, 64)],
                                 osem[b])

        return carry

    a_outer = (NBLK_FULL // NW + 1 + A_NBUF - 1) // A_NBUF
    lax.fori_loop(0, a_outer, outer, 0)
    for b in range(A_NBUF):
        pltpu.make_async_copy(
            outs[b], packed_hbm.at[pl.ds(0, 64)], osem[b]).wait()


_repack = pl.kernel(
    _repack_body,
    out_type=jax.ShapeDtypeStruct((PACK_ROWS, 128), jnp.float32),
    mesh=plsc.VectorSubcoreMesh(
        core_axis_name="c", subcore_axis_name="s",
        num_cores=NC, num_subcores=NS),
    compiler_params=pltpu.CompilerParams(use_tc_tiling_on_sc=True,
                                         needs_layout_passes=False),
    scratch_types=(
        [pltpu.VMEM((D_MODEL, 128), jnp.float32) for _ in range(A_NBUF)]
        + [pltpu.VMEM((64, 128), jnp.float32) for _ in range(A_NBUF)]
        + [pltpu.SemaphoreType.DMA for _ in range(2 * A_NBUF)]
    ),
)


def _gather_body(xt_hbm, packed_hbm, tail_hbm, out_hbm, tailb, *scratch):
    ibufs = scratch[:B_NBUF]
    jbufs = scratch[B_NBUF:2 * B_NBUF]
    rows = scratch[2 * B_NBUF:3 * B_NBUF]
    slabs = scratch[3 * B_NBUF:4 * B_NBUF]
    isem = scratch[4 * B_NBUF:5 * B_NBUF]
    gsem = scratch[5 * B_NBUF:6 * B_NBUF]
    osem = scratch[6 * B_NBUF:7 * B_NBUF]

    wid = lax.axis_index("s") * NC + lax.axis_index("c")
    b0 = pl.multiple_of(wid * 128, 128)
    lane = lax.iota(jnp.int32, 16)

    pltpu.sync_copy(tail_hbm, tailb)

    def outer(o, carry):
        for b in range(B_NBUF):
            t = o * B_NBUF + b
            pltpu.async_copy(
                xt_hbm.at[t, pl.ds(b0, 128)], ibufs[b], isem[b])

        for b in range(B_NBUF):
            t = o * B_NBUF + b
            pltpu.make_async_copy(
                xt_hbm.at[t, pl.ds(b0, 128)], ibufs[b], isem[b]).wait()

            def shift_row(k, c2, b=b):
                jbufs[b][pl.ds(16 * k, 16)] = jnp.minimum(
                    lax.shift_right_logical(ibufs[b][pl.ds(16 * k, 16)], 1),
                    PACK_ROWS - 33)
                return c2

            lax.fori_loop(0, 8, shift_row, 0)
            pltpu.async_copy(packed_hbm.at[jbufs[b]], rows[b], gsem[b])

        for b in range(B_NBUF):
            t = o * B_NBUF + b

            @pl.when(o > 0)
            def _drain(b=b):
                pltpu.make_async_copy(
                    slabs[b], out_hbm.at[0, pl.ds(0, D_MODEL), pl.ds(0, 128)],
                    osem[b]).wait()

            pltpu.make_async_copy(
                packed_hbm.at[jbufs[b]], rows[b], gsem[b]).wait()

            for k in range(8):
                iv = ibufs[b][pl.ds(16 * k, 16)]
                par = (iv & 1) * 64
                row_ids = lane + (16 * k)

                def trans_d(d, c2, b=b, par=par, row_ids=row_ids, k=k):
                    vals = plsc.load_gather(rows[b], [row_ids, par + d])
                    slabs[b][d, pl.ds(16 * k, 16)] = vals
                    return c2

                lax.fori_loop(0, D_MODEL, trans_d, 0)

                tmask = iv >= TAIL_V0

                @pl.when(jnp.any(tmask))
                def _fixup(b=b, k=k, iv=iv, tmask=tmask):
                    tidx = jnp.maximum(iv - TAIL_V0, 0)

                    def fix_d(d, c2, b=b, k=k):
                        tv = plsc.load_gather(tailb, [tidx, (lane * 0) + d])
                        cur = slabs[b][d, pl.ds(16 * k, 16)]
                        slabs[b][d, pl.ds(16 * k, 16)] = jnp.where(
                            tmask, tv, cur)
                        return c2

                    lax.fori_loop(0, D_MODEL, fix_d, 0)

            pltpu.async_copy(
                slabs[b],
                out_hbm.at[t, pl.ds(0, D_MODEL), pl.ds(b0, 128)], osem[b])
        return carry

    lax.fori_loop(0, SEQ // B_NBUF, outer, 0)
    for b in range(B_NBUF):
        pltpu.make_async_copy(
            slabs[b], out_hbm.at[0, pl.ds(0, D_MODEL), pl.ds(0, 128)],
            osem[b]).wait()


_gather = pl.kernel(
    _gather_body,
    out_type=jax.ShapeDtypeStruct((SEQ, D_MODEL, BATCH), jnp.float32),
    mesh=plsc.VectorSubcoreMesh(
        core_axis_name="c", subcore_axis_name="s",
        num_cores=NC, num_subcores=NS),
    compiler_params=pltpu.CompilerParams(use_tc_tiling_on_sc=True,
                                         needs_layout_passes=False),
    scratch_types=(
        [pltpu.VMEM((64, 128), jnp.float32)]
        + [pltpu.VMEM((128,), jnp.int32) for _ in range(B_NBUF)]
        + [pltpu.VMEM((128,), jnp.int32) for _ in range(B_NBUF)]
        + [pltpu.VMEM((128, 128), jnp.float32) for _ in range(B_NBUF)]
        + [pltpu.VMEM((D_MODEL, 128), jnp.float32) for _ in range(B_NBUF)]
        + [pltpu.SemaphoreType.DMA for _ in range(3 * B_NBUF)]
    ),
)


def kernel(x, table):
    packed = _repack(table.T)
    tail = jnp.pad(table[TAIL_V0:] * SCALE, ((0, 0), (0, 64)))
    out_phys = _gather(x.T, packed, tail)
    return out_phys.transpose(2, 0, 1)


# speedup vs baseline: 1.7955x; 1.7955x over previous
import math

import jax
import jax.numpy as jnp
from jax import lax
from jax.experimental import pallas as pl
from jax.experimental.pallas import tpu as pltpu
from jax.experimental.pallas import tpu_sc as plsc

D_MODEL = 64
VOCAB = 1000000
BATCH = 4096
SEQ = 200
SCALE = math.sqrt(D_MODEL)

NC, NS, L = 2, 16, 16
NW = NC * NS

NBLK_FULL = VOCAB // 128
TAIL_V0 = NBLK_FULL * 128
A_NBUF = 4
B_NBUF = 4


def _repack_body(tt_hbm, tail_hbm, packed_hbm, *scratch):
    bufs = scratch[:A_NBUF]
    outs = scratch[A_NBUF:2 * A_NBUF]
    isem = scratch[2 * A_NBUF:3 * A_NBUF]
    osem = scratch[3 * A_NBUF:4 * A_NBUF]

    wid = lax.axis_index("s") * NC + lax.axis_index("c")
    nblk_w = (NBLK_FULL - wid + NW - 1) // NW
    lane = lax.iota(jnp.int32, 16)

    @pl.when(wid == 0)
    def _tail():
        pltpu.sync_copy(tail_hbm, packed_hbm.at[pl.ds(TAIL_V0, 64)])

    def outer(o, carry):
        for b in range(A_NBUF):
            n = o * A_NBUF + b

            @pl.when(n < nblk_w)
            def _start(b=b, n=n):
                @pl.when(o > 0)
                def _drain(b=b):
                    pltpu.make_async_copy(
                        outs[b],
                        packed_hbm.at[pl.ds(0, 128)],
                        osem[b]).wait()

                v0 = pl.multiple_of((wid + n * NW) * 128, 128)
                pltpu.async_copy(
                    tt_hbm.at[pl.ds(0, D_MODEL), pl.ds(v0, 128)],
                    bufs[b], isem[b])

        for b in range(A_NBUF):
            n = o * A_NBUF + b

            @pl.when(n < nblk_w)
            def _work(b=b, n=n):
                v0 = pl.multiple_of((wid + n * NW) * 128, 128)
                pltpu.make_async_copy(
                    tt_hbm.at[pl.ds(0, D_MODEL), pl.ds(v0, 128)],
                    bufs[b], isem[b]).wait()

                @plsc.parallel_loop(0, 128, carry=lane * 0)
                def _pack(i, colv, b=b):
                    for m in range(4):
                        vals = plsc.load_gather(
                            bufs[b], [lane + 16 * m, colv])
                        outs[b][i, pl.ds(16 * m, 16)] = vals * SCALE
                    return colv + 1

                pltpu.async_copy(
                    outs[b],
                    packed_hbm.at[pl.ds(v0, 128)],
                    osem[b])

        return carry

    a_outer = (NBLK_FULL // NW + 1 + A_NBUF - 1) // A_NBUF
    lax.fori_loop(0, a_outer, outer, 0)
    for b in range(A_NBUF):
        pltpu.make_async_copy(
            outs[b], packed_hbm.at[pl.ds(0, 128)],
            osem[b]).wait()


_repack = pl.kernel(
    _repack_body,
    out_type=jax.ShapeDtypeStruct((VOCAB, 128), jnp.float32),
    mesh=plsc.VectorSubcoreMesh(
        core_axis_name="c", subcore_axis_name="s",
        num_cores=NC, num_subcores=NS),
    compiler_params=pltpu.CompilerParams(use_tc_tiling_on_sc=True,
                                         needs_layout_passes=False),
    scratch_types=(
        [pltpu.VMEM((D_MODEL, 128), jnp.float32) for _ in range(A_NBUF)]
        + [pltpu.VMEM((128, 128), jnp.float32) for _ in range(A_NBUF)]
        + [pltpu.SemaphoreType.DMA for _ in range(2 * A_NBUF)]
    ),
)


def _gather_body(xt_hbm, packed_hbm, out_hbm, *scratch):
    ibufs = scratch[:B_NBUF]
    rows = scratch[B_NBUF:2 * B_NBUF]
    slabs = scratch[2 * B_NBUF:3 * B_NBUF]
    isem = scratch[3 * B_NBUF:4 * B_NBUF]
    gsem = scratch[4 * B_NBUF:5 * B_NBUF]
    osem = scratch[5 * B_NBUF:6 * B_NBUF]

    wid = lax.axis_index("s") * NC + lax.axis_index("c")
    b0 = pl.multiple_of(wid * 128, 128)
    lane = lax.iota(jnp.int32, 16)

    def outer(o, carry):
        for b in range(B_NBUF):
            t = o * B_NBUF + b
            pltpu.async_copy(
                xt_hbm.at[t, pl.ds(b0, 128)], ibufs[b], isem[b])

        for b in range(B_NBUF):
            t = o * B_NBUF + b
            pltpu.make_async_copy(
                xt_hbm.at[t, pl.ds(b0, 128)], ibufs[b], isem[b]).wait()
            pltpu.async_copy(packed_hbm.at[ibufs[b]], rows[b], gsem[b])

        for b in range(B_NBUF):
            t = o * B_NBUF + b

            @pl.when(o > 0)
            def _drain(b=b):
                pltpu.make_async_copy(
                    slabs[b], out_hbm.at[0, pl.ds(0, D_MODEL), pl.ds(0, 128)],
                    osem[b]).wait()

            pltpu.make_async_copy(
                packed_hbm.at[ibufs[b]], rows[b], gsem[b]).wait()

            @plsc.parallel_loop(0, D_MODEL, carry=lane * 0)
            def _trans(d, colv, b=b):
                for k in range(8):
                    vals = plsc.load_gather(rows[b], [lane + 16 * k, colv])
                    slabs[b][d, pl.ds(16 * k, 16)] = vals
                return colv + 1

            pltpu.async_copy(
                slabs[b],
                out_hbm.at[t, pl.ds(0, D_MODEL), pl.ds(b0, 128)], osem[b])
        return carry

    lax.fori_loop(0, SEQ // B_NBUF, outer, 0)
    for b in range(B_NBUF):
        pltpu.make_async_copy(
            slabs[b], out_hbm.at[0, pl.ds(0, D_MODEL), pl.ds(0, 128)],
            osem[b]).wait()


_gather = pl.kernel(
    _gather_body,
    out_type=jax.ShapeDtypeStruct((SEQ, D_MODEL, BATCH), jnp.float32),
    mesh=plsc.VectorSubcoreMesh(
        core_axis_name="c", subcore_axis_name="s",
        num_cores=NC, num_subcores=NS),
    compiler_params=pltpu.CompilerParams(use_tc_tiling_on_sc=True,
                                         needs_layout_passes=False),
    scratch_types=(
        [pltpu.VMEM((128,), jnp.int32) for _ in range(B_NBUF)]
        + [pltpu.VMEM((128, 128), jnp.float32) for _ in range(B_NBUF)]
        + [pltpu.VMEM((D_MODEL, 128), jnp.float32) for _ in range(B_NBUF)]
        + [pltpu.SemaphoreType.DMA for _ in range(3 * B_NBUF)]
    ),
)


def kernel(x, table):
    tail = jnp.pad(table[TAIL_V0:] * SCALE, ((0, 0), (0, 64)))
    packed = _repack(table.T, tail)
    out_phys = _gather(x.T, packed)
    return out_phys.transpose(2, 0, 1)


# speedup vs baseline: 2.3700x; 1.3200x over previous
import math

import jax
import jax.numpy as jnp
from jax import lax
from jax.experimental import pallas as pl
from jax.experimental.pallas import tpu as pltpu
from jax.experimental.pallas import tpu_sc as plsc

D_MODEL = 64
VOCAB = 1000000
BATCH = 4096
SEQ = 200
SCALE = math.sqrt(D_MODEL)

NC, NS, L = 2, 16, 16
NW = NC * NS

NBLK_FULL = VOCAB // 128
TAIL_V0 = NBLK_FULL * 128
A_NBUF = 4
B_NBUF = 4


def _repack_body(tt_hbm, tail_hbm, packed_hbm, *scratch):
    bufs = scratch[:A_NBUF]
    outs = scratch[A_NBUF:2 * A_NBUF]
    isem = scratch[2 * A_NBUF:3 * A_NBUF]
    osem = scratch[3 * A_NBUF:4 * A_NBUF]

    wid = lax.axis_index("s") * NC + lax.axis_index("c")
    nblk_w = (NBLK_FULL - wid + NW - 1) // NW
    lane = lax.iota(jnp.int32, 16)
    diag = [(lane + j) & 15 for j in range(16)]

    @pl.when(wid == 0)
    def _tail():
        pltpu.sync_copy(tail_hbm, packed_hbm.at[pl.ds(TAIL_V0, 64)])

    def outer(o, carry):
        for b in range(A_NBUF):
            n = o * A_NBUF + b

            @pl.when(n < nblk_w)
            def _start(b=b, n=n):
                @pl.when(o > 0)
                def _drain(b=b):
                    pltpu.make_async_copy(
                        outs[b],
                        packed_hbm.at[pl.ds(0, 128)],
                        osem[b]).wait()

                v0 = pl.multiple_of((wid + n * NW) * 128, 128)
                pltpu.async_copy(
                    tt_hbm.at[pl.ds(0, D_MODEL), pl.ds(v0, 128)],
                    bufs[b], isem[b])

        for b in range(A_NBUF):
            n = o * A_NBUF + b

            @pl.when(n < nblk_w)
            def _work(b=b, n=n):
                v0 = pl.multiple_of((wid + n * NW) * 128, 128)
                pltpu.make_async_copy(
                    tt_hbm.at[pl.ds(0, D_MODEL), pl.ds(v0, 128)],
                    bufs[b], isem[b]).wait()

                for g in range(4):
                    @plsc.parallel_loop(0, 8)
                    def _pack(h, b=b, g=g):
                        ivec = 16 * h + lane
                        for j in range(16):
                            dvec = 16 * g + diag[j]
                            vals = plsc.load_gather(bufs[b], [dvec, ivec])
                            plsc.store_scatter(
                                outs[b], [ivec, dvec], vals * SCALE)

                pltpu.async_copy(
                    outs[b],
                    packed_hbm.at[pl.ds(v0, 128)],
                    osem[b])

        return carry

    a_outer = (NBLK_FULL // NW + 1 + A_NBUF - 1) // A_NBUF
    lax.fori_loop(0, a_outer, outer, 0)
    for b in range(A_NBUF):
        pltpu.make_async_copy(
            outs[b], packed_hbm.at[pl.ds(0, 128)],
            osem[b]).wait()


_repack = pl.kernel(
    _repack_body,
    out_type=jax.ShapeDtypeStruct((VOCAB, 128), jnp.float32),
    mesh=plsc.VectorSubcoreMesh(
        core_axis_name="c", subcore_axis_name="s",
        num_cores=NC, num_subcores=NS),
    compiler_params=pltpu.CompilerParams(use_tc_tiling_on_sc=True,
                                         needs_layout_passes=False),
    scratch_types=(
        [pltpu.VMEM((D_MODEL, 128), jnp.float32) for _ in range(A_NBUF)]
        + [pltpu.VMEM((128, 128), jnp.float32) for _ in range(A_NBUF)]
        + [pltpu.SemaphoreType.DMA for _ in range(2 * A_NBUF)]
    ),
)


def _gather_body(xt_hbm, packed_hbm, out_hbm, *scratch):
    ibufs = scratch[:B_NBUF]
    rows = scratch[B_NBUF:2 * B_NBUF]
    slabs = scratch[2 * B_NBUF:3 * B_NBUF]
    isem = scratch[3 * B_NBUF:4 * B_NBUF]
    gsem = scratch[4 * B_NBUF:5 * B_NBUF]
    osem = scratch[5 * B_NBUF:6 * B_NBUF]

    wid = lax.axis_index("s") * NC + lax.axis_index("c")
    b0 = pl.multiple_of(wid * 128, 128)
    lane = lax.iota(jnp.int32, 16)
    diag = [(lane + j) & 15 for j in range(16)]

    def outer(o, carry):
        for b in range(B_NBUF):
            t = o * B_NBUF + b
            pltpu.async_copy(
                xt_hbm.at[t, pl.ds(b0, 128)], ibufs[b], isem[b])

        for b in range(B_NBUF):
            t = o * B_NBUF + b
            pltpu.make_async_copy(
                xt_hbm.at[t, pl.ds(b0, 128)], ibufs[b], isem[b]).wait()
            pltpu.async_copy(packed_hbm.at[ibufs[b]], rows[b], gsem[b])

        for b in range(B_NBUF):
            t = o * B_NBUF + b

            @pl.when(o > 0)
            def _drain(b=b):
                pltpu.make_async_copy(
                    slabs[b], out_hbm.at[0, pl.ds(0, D_MODEL), pl.ds(0, 128)],
                    osem[b]).wait()

            pltpu.make_async_copy(
                packed_hbm.at[ibufs[b]], rows[b], gsem[b]).wait()

            for g in range(4):
                @plsc.parallel_loop(0, 8)
                def _trans(h, b=b, g=g):
                    ivec = 16 * h + lane
                    for j in range(16):
                        dvec = 16 * g + diag[j]
                        vals = plsc.load_gather(rows[b], [ivec, dvec])
                        plsc.store_scatter(slabs[b], [dvec, ivec], vals)

            pltpu.async_copy(
                slabs[b],
                out_hbm.at[t, pl.ds(0, D_MODEL), pl.ds(b0, 128)], osem[b])
        return carry

    lax.fori_loop(0, SEQ // B_NBUF, outer, 0)
    for b in range(B_NBUF):
        pltpu.make_async_copy(
            slabs[b], out_hbm.at[0, pl.ds(0, D_MODEL), pl.ds(0, 128)],
            osem[b]).wait()


_gather = pl.kernel(
    _gather_body,
    out_type=jax.ShapeDtypeStruct((SEQ, D_MODEL, BATCH), jnp.float32),
    mesh=plsc.VectorSubcoreMesh(
        core_axis_name="c", subcore_axis_name="s",
        num_cores=NC, num_subcores=NS),
    compiler_params=pltpu.CompilerParams(use_tc_tiling_on_sc=True,
                                         needs_layout_passes=False),
    scratch_types=(
        [pltpu.VMEM((128,), jnp.int32) for _ in range(B_NBUF)]
        + [pltpu.VMEM((128, 128), jnp.float32) for _ in range(B_NBUF)]
        + [pltpu.VMEM((D_MODEL, 128), jnp.float32) for _ in range(B_NBUF)]
        + [pltpu.SemaphoreType.DMA for _ in range(3 * B_NBUF)]
    ),
)


def kernel(x, table):
    tail = jnp.pad(table[TAIL_V0:] * SCALE, ((0, 0), (0, 64)))
    packed = _repack(table.T, tail)
    out_phys = _gather(x.T, packed)
    return out_phys.transpose(2, 0, 1)
